# baseline (speedup 1.0000x reference)
import functools

import jax
import jax.numpy as jnp
from jax import lax
from jax.experimental import pallas as pl
from jax.experimental.pallas import tpu as pltpu
from jax.experimental.pallas import tpu_sc as plsc

_NUM_CLASSES = 2
_CLIPMIN = 0.05
_CLIPMAX = 0.95

_ROWS = 8192
_COLS = 4096
_TOTAL = _ROWS * _COLS

_NC = 2
_NS = 16
_NW = _NC * _NS
_R_SC = 2560
_ROWS_W = _R_SC // _NW
_CH_ROWS = 4
_NCH = _ROWS_W // _CH_ROWS
_VPR = _COLS // 16
_BLK = 512


def _sc_reduce_chunk(lab_bufs, m_bufs, accs):
    def inner(i, accs):
        new = []
        for r in range(_CH_ROWS):
            lab = lab_bufs[r][pl.ds(i * 16, 16)]
            m = m_bufs[r][pl.ds(i * 16, 16)]
            am, c1, cs = accs[r]
            selm = m > 0.0
            ones = jnp.where(selm, 1.0, 0.0)
            am = am + m
            cs = cs + ones
            c1 = c1 + ones * lab.astype(jnp.float32)
            new.append((am, c1, cs))
        return tuple(new)

    return lax.fori_loop(0, _VPR, inner, accs)


def _sc_pass1_body(labels_hbm, mask_hbm, out_hbm, lab_buf, m_buf, outv,
                   sl0, sl1, sm0, sm1):
    wid = lax.axis_index("s") * _NC + lax.axis_index("c")
    base = wid * _ROWS_W
    sem_l = (sl0, sl1)
    sem_m = (sm0, sm1)

    pltpu.async_copy(labels_hbm.at[pl.ds(base, _CH_ROWS)], lab_buf.at[0], sem_l[0])
    pltpu.async_copy(mask_hbm.at[pl.ds(base, _CH_ROWS)], m_buf.at[0], sem_m[0])

    zero = jnp.zeros((16,), jnp.float32)
    accs0 = tuple((zero, zero, zero) for _ in range(_CH_ROWS))

    def body2(p, accs):
        for b in range(2):
            k = 2 * p + b
            nb = 1 - b

            @pl.when(k + 1 < _NCH)
            def _start_next():
                off = base + (k + 1) * _CH_ROWS
                pltpu.async_copy(labels_hbm.at[pl.ds(off, _CH_ROWS)],
                                 lab_buf.at[nb], sem_l[nb])
                pltpu.async_copy(mask_hbm.at[pl.ds(off, _CH_ROWS)],
                                 m_buf.at[nb], sem_m[nb])

            pltpu.make_async_copy(labels_hbm.at[pl.ds(0, _CH_ROWS)],
                                  lab_buf.at[b], sem_l[b]).wait()
            pltpu.make_async_copy(mask_hbm.at[pl.ds(0, _CH_ROWS)],
                                  m_buf.at[b], sem_m[b]).wait()
            accs = _sc_reduce_chunk(
                [lab_buf.at[b, r] for r in range(_CH_ROWS)],
                [m_buf.at[b, r] for r in range(_CH_ROWS)], accs)
        return accs

    accs = lax.fori_loop(0, _NCH // 2, body2, accs0)

    am = accs[0][0] + accs[1][0] + accs[2][0] + accs[3][0]
    c1 = accs[0][1] + accs[1][1] + accs[2][1] + accs[3][1]
    cs = accs[0][2] + accs[1][2] + accs[2][2] + accs[3][2]
    outv[pl.ds(0, 16)] = am
    outv[pl.ds(16, 16)] = c1
    outv[pl.ds(32, 16)] = cs
    pltpu.sync_copy(outv, out_hbm.at[pl.ds(wid * 48, 48)])


_sc_pass1 = functools.partial(
    pl.kernel,
    mesh=plsc.VectorSubcoreMesh(core_axis_name="c", subcore_axis_name="s"),
    out_type=jax.ShapeDtypeStruct((_NW * 48,), jnp.float32),
    scratch_types=[
        pltpu.VMEM((2, _CH_ROWS, _COLS), jnp.int32),
        pltpu.VMEM((2, _CH_ROWS, _COLS), jnp.float32),
        pltpu.VMEM((48,), jnp.float32),
        pltpu.SemaphoreType.DMA,
        pltpu.SemaphoreType.DMA,
        pltpu.SemaphoreType.DMA,
        pltpu.SemaphoreType.DMA,
    ],
)(_sc_pass1_body)


def _tc_pass1_body(labels_ref, mask_ref, acc_ref, packed_ref):
    i = pl.program_id(0)

    @pl.when(i == 0)
    def _init():
        acc_ref[...] = jnp.zeros_like(acc_ref)

    m = mask_ref[...]
    lab = labels_ref[...]
    sel = (m > 0.0).astype(jnp.float32)
    s_mask = jnp.sum(m)
    c1 = jnp.sum(sel * lab.astype(jnp.float32))
    csel = jnp.sum(sel)
    lane = lax.broadcasted_iota(jnp.int32, (1, 128), 1)
    pv = (jnp.where(lane == 0, s_mask, 0.0)
          + jnp.where(lane == 1, c1, 0.0)
          + jnp.where(lane == 2, csel, 0.0))
    acc_ref[...] += pv
    lab3 = lab.reshape(_BLK // 32, 32, _COLS)
    k = lax.broadcasted_iota(jnp.int32, (_BLK // 32, 32, _COLS), 1)
    packed_ref[...] = jnp.sum(lab3 << k, axis=1)


def _weights(acc_ref, tc_acc_ref):
    acc = acc_ref[...]
    row = lax.broadcasted_iota(jnp.int32, (12, 128), 0)
    col = lax.broadcasted_iota(jnp.int32, (12, 128), 1)
    lane = (row * 128 + col) % 48
    tc_acc = tc_acc_ref[...]
    masked_in = jnp.sum(jnp.where(lane < 16, acc, 0.0)) + tc_acc[0, 0]
    c1 = jnp.sum(jnp.where((lane >= 16) & (lane < 32), acc, 0.0)) + tc_acc[0, 1]
    csel = jnp.sum(jnp.where(lane >= 32, acc, 0.0)) + tc_acc[0, 2]
    c0 = csel - c1

    inv_n = 1.0 / float(_NUM_CLASSES)

    def weight(c):
        frac = jnp.where(masked_in > 0.0, c / masked_in, 0.0)
        frac = jnp.clip(frac, _CLIPMIN, _CLIPMAX)
        w = inv_n / frac
        return jnp.where(c > 0.0, w, 0.0)

    return weight(c0), weight(c1)


def _pass2a_body(acc_ref, tc_acc_ref, labels_ref, mask_ref, out_ref):
    w0, w1 = _weights(acc_ref, tc_acc_ref)
    m = mask_ref[...]
    lab = labels_ref[...]
    out_ref[...] = m * jnp.where(lab == 1, w1, w0)


def _pass2b_body(acc_ref, tc_acc_ref, packed_ref, mask_ref, prev_ref, out_ref):
    del prev_ref
    w0, w1 = _weights(acc_ref, tc_acc_ref)
    m = mask_ref[...]
    packed = packed_ref[...]
    p3 = jnp.broadcast_to(packed[:, None, :], (_BLK // 32, 32, _COLS))
    k = lax.broadcasted_iota(jnp.int32, (_BLK // 32, 32, _COLS), 1)
    lab = ((p3 >> k) & 1).reshape(_BLK, _COLS)
    out_ref[...] = m * jnp.where(lab == 1, w1, w0)


@jax.jit
def kernel(labels, mask):
    acc = _sc_pass1(labels, mask)
    acc = acc.reshape(12, 128)

    tc_grid = (_ROWS - _R_SC) // _BLK
    blk0 = _R_SC // _BLK
    tc_acc, packed = pl.pallas_call(
        _tc_pass1_body,
        grid=(tc_grid,),
        in_specs=[
            pl.BlockSpec((_BLK, _COLS), lambda i: (i + blk0, 0)),
            pl.BlockSpec((_BLK, _COLS), lambda i: (i + blk0, 0)),
        ],
        out_specs=[
            pl.BlockSpec((1, 128), lambda i: (0, 0)),
            pl.BlockSpec((_BLK // 32, _COLS), lambda i: (i, 0)),
        ],
        out_shape=[
            jax.ShapeDtypeStruct((1, 128), jnp.float32),
            jax.ShapeDtypeStruct(((_ROWS - _R_SC) // 32, _COLS), jnp.int32),
        ],
    )(labels, mask)

    out_a = pl.pallas_call(
        _pass2a_body,
        grid=(blk0,),
        in_specs=[
            pl.BlockSpec((12, 128), lambda i: (0, 0)),
            pl.BlockSpec((1, 128), lambda i: (0, 0)),
            pl.BlockSpec((_BLK, _COLS), lambda i: (i, 0)),
            pl.BlockSpec((_BLK, _COLS), lambda i: (i, 0)),
        ],
        out_specs=pl.BlockSpec((_BLK, _COLS), lambda i: (i, 0)),
        out_shape=jax.ShapeDtypeStruct((_ROWS, _COLS), jnp.float32),
    )(acc, tc_acc, labels, mask)

    out = pl.pallas_call(
        _pass2b_body,
        grid=(tc_grid,),
        in_specs=[
            pl.BlockSpec((12, 128), lambda i: (0, 0)),
            pl.BlockSpec((1, 128), lambda i: (0, 0)),
            pl.BlockSpec((_BLK // 32, _COLS), lambda i: (i, 0)),
            pl.BlockSpec((_BLK, _COLS), lambda i: (i + blk0, 0)),
            pl.BlockSpec((8, 128), lambda i: (0, 0)),
        ],
        out_specs=pl.BlockSpec((_BLK, _COLS), lambda i: (i + blk0, 0)),
        out_shape=jax.ShapeDtypeStruct((_ROWS, _COLS), jnp.float32),
        input_output_aliases={4: 0},
    )(acc, tc_acc, packed, mask, out_a)
    return out


# speedup vs baseline: 1.0539x; 1.0539x over previous
import functools

import jax
import jax.numpy as jnp
from jax import lax
from jax.experimental import pallas as pl
from jax.experimental.pallas import tpu as pltpu
from jax.experimental.pallas import tpu_sc as plsc

_NUM_CLASSES = 2
_CLIPMIN = 0.05
_CLIPMAX = 0.95

_ROWS = 8192
_COLS = 4096
_TOTAL = _ROWS * _COLS

_NC = 2
_NS = 16
_NW = _NC * _NS
_R_SC = 1024
_ROWS_W = _R_SC // _NW
_CH_ROWS = 4
_NCH = _ROWS_W // _CH_ROWS
_VPR = _COLS // 16
_BLK = 512


def _sc_reduce_chunk(lab_bufs, m_bufs, pacc, k, accs):
    shift = 4 * k

    def inner(i, accs):
        new = []
        labs = []
        for r in range(_CH_ROWS):
            lab = lab_bufs[r][pl.ds(i * 16, 16)]
            m = m_bufs[r][pl.ds(i * 16, 16)]
            am, c1, cs = accs[r]
            selm = m > 0.0
            ones = jnp.where(selm, 1.0, 0.0)
            am = am + m
            cs = cs + ones
            c1 = c1 + ones * lab.astype(jnp.float32)
            labs.append(lab)
            new.append((am, c1, cs))
        nib = (labs[0] | (labs[1] << 1) | (labs[2] << 2) | (labs[3] << 3))
        pacc[pl.ds(i * 16, 16)] = pacc[pl.ds(i * 16, 16)] | (nib << shift)
        return tuple(new)

    return lax.fori_loop(0, _VPR, inner, accs)


def _sc_pass1_body(labels_hbm, mask_hbm, out_hbm, packed_hbm, lab_buf, m_buf,
                   outv, pacc, sl0, sl1, sm0, sm1):
    wid = lax.axis_index("s") * _NC + lax.axis_index("c")
    base = wid * _ROWS_W
    sem_l = (sl0, sl1)
    sem_m = (sm0, sm1)

    pltpu.async_copy(labels_hbm.at[pl.ds(base, _CH_ROWS)], lab_buf.at[0], sem_l[0])
    pltpu.async_copy(mask_hbm.at[pl.ds(base, _CH_ROWS)], m_buf.at[0], sem_m[0])

    zero_i = jnp.zeros((16,), jnp.int32)

    def zero_body(i, _):
        pacc[pl.ds(i * 16, 16)] = zero_i
        return 0

    lax.fori_loop(0, _VPR, zero_body, 0)

    zero = jnp.zeros((16,), jnp.float32)
    accs0 = tuple((zero, zero, zero) for _ in range(_CH_ROWS))

    def body2(p, accs):
        for b in range(2):
            k = 2 * p + b
            nb = 1 - b

            @pl.when(k + 1 < _NCH)
            def _start_next():
                off = base + (k + 1) * _CH_ROWS
                pltpu.async_copy(labels_hbm.at[pl.ds(off, _CH_ROWS)],
                                 lab_buf.at[nb], sem_l[nb])
                pltpu.async_copy(mask_hbm.at[pl.ds(off, _CH_ROWS)],
                                 m_buf.at[nb], sem_m[nb])

            pltpu.make_async_copy(labels_hbm.at[pl.ds(0, _CH_ROWS)],
                                  lab_buf.at[b], sem_l[b]).wait()
            pltpu.make_async_copy(mask_hbm.at[pl.ds(0, _CH_ROWS)],
                                  m_buf.at[b], sem_m[b]).wait()
            accs = _sc_reduce_chunk(
                [lab_buf.at[b, r] for r in range(_CH_ROWS)],
                [m_buf.at[b, r] for r in range(_CH_ROWS)], pacc, k, accs)
        return accs

    accs = lax.fori_loop(0, _NCH // 2, body2, accs0)

    am = accs[0][0] + accs[1][0] + accs[2][0] + accs[3][0]
    c1 = accs[0][1] + accs[1][1] + accs[2][1] + accs[3][1]
    cs = accs[0][2] + accs[1][2] + accs[2][2] + accs[3][2]
    outv[pl.ds(0, 16)] = am
    outv[pl.ds(16, 16)] = c1
    outv[pl.ds(32, 16)] = cs
    pltpu.sync_copy(outv, out_hbm.at[pl.ds(wid * 48, 48)])
    pltpu.sync_copy(pacc, packed_hbm.at[pl.ds(wid * _COLS, _COLS)])


_sc_pass1 = functools.partial(
    pl.kernel,
    mesh=plsc.VectorSubcoreMesh(core_axis_name="c", subcore_axis_name="s"),
    out_type=[
        jax.ShapeDtypeStruct((_NW * 48,), jnp.float32),
        jax.ShapeDtypeStruct((_R_SC // 32 * _COLS,), jnp.int32),
    ],
    scratch_types=[
        pltpu.VMEM((2, _CH_ROWS, _COLS), jnp.int32),
        pltpu.VMEM((2, _CH_ROWS, _COLS), jnp.float32),
        pltpu.VMEM((48,), jnp.float32),
        pltpu.VMEM((_COLS,), jnp.int32),
        pltpu.SemaphoreType.DMA,
        pltpu.SemaphoreType.DMA,
        pltpu.SemaphoreType.DMA,
        pltpu.SemaphoreType.DMA,
    ],
)(_sc_pass1_body)


def _tc_pass1_body(labels_ref, mask_ref, acc_ref, packed_ref):
    i = pl.program_id(0)

    @pl.when(i == 0)
    def _init():
        acc_ref[...] = jnp.zeros_like(acc_ref)

    m = mask_ref[...]
    lab = labels_ref[...]
    sel = (m > 0.0).astype(jnp.float32)
    s_mask = jnp.sum(m)
    c1 = jnp.sum(sel * lab.astype(jnp.float32))
    csel = jnp.sum(sel)
    lane = lax.broadcasted_iota(jnp.int32, (1, 128), 1)
    pv = (jnp.where(lane == 0, s_mask, 0.0)
          + jnp.where(lane == 1, c1, 0.0)
          + jnp.where(lane == 2, csel, 0.0))
    acc_ref[...] += pv
    lab3 = lab.reshape(_BLK // 32, 32, _COLS)
    k = lax.broadcasted_iota(jnp.int32, (_BLK // 32, 32, _COLS), 1)
    packed_ref[...] = jnp.sum(lab3 << k, axis=1)


def _weights(acc_ref, tc_acc_ref):
    acc = acc_ref[...]
    row = lax.broadcasted_iota(jnp.int32, (12, 128), 0)
    col = lax.broadcasted_iota(jnp.int32, (12, 128), 1)
    lane = (row * 128 + col) % 48
    tc_acc = tc_acc_ref[...]
    masked_in = jnp.sum(jnp.where(lane < 16, acc, 0.0)) + tc_acc[0, 0]
    c1 = jnp.sum(jnp.where((lane >= 16) & (lane < 32), acc, 0.0)) + tc_acc[0, 1]
    csel = jnp.sum(jnp.where(lane >= 32, acc, 0.0)) + tc_acc[0, 2]
    c0 = csel - c1

    inv_n = 1.0 / float(_NUM_CLASSES)

    def weight(c):
        frac = jnp.where(masked_in > 0.0, c / masked_in, 0.0)
        frac = jnp.clip(frac, _CLIPMIN, _CLIPMAX)
        w = inv_n / frac
        return jnp.where(c > 0.0, w, 0.0)

    return weight(c0), weight(c1)


def _unpacked_scale(packed_ref, mask_ref, out_ref, w0, w1):
    m = mask_ref[...]
    packed = packed_ref[...]
    p3 = jnp.broadcast_to(packed[:, None, :], (_BLK // 32, 32, _COLS))
    k = lax.broadcasted_iota(jnp.int32, (_BLK // 32, 32, _COLS), 1)
    lab = ((p3 >> k) & 1).reshape(_BLK, _COLS)
    out_ref[...] = m * jnp.where(lab == 1, w1, w0)


def _pass2a_body(acc_ref, tc_acc_ref, packed_ref, mask_ref, out_ref):
    w0, w1 = _weights(acc_ref, tc_acc_ref)
    _unpacked_scale(packed_ref, mask_ref, out_ref, w0, w1)


def _pass2b_body(acc_ref, tc_acc_ref, packed_ref, mask_ref, prev_ref, out_ref):
    del prev_ref
    w0, w1 = _weights(acc_ref, tc_acc_ref)
    _unpacked_scale(packed_ref, mask_ref, out_ref, w0, w1)


@jax.jit
def kernel(labels, mask):
    acc, packed_sc = _sc_pass1(labels, mask)
    acc = acc.reshape(12, 128)
    packed_sc = packed_sc.reshape(_R_SC // 32, _COLS)

    tc_grid = (_ROWS - _R_SC) // _BLK
    blk0 = _R_SC // _BLK
    tc_acc, packed = pl.pallas_call(
        _tc_pass1_body,
        grid=(tc_grid,),
        in_specs=[
            pl.BlockSpec((_BLK, _COLS), lambda i: (i + blk0, 0)),
            pl.BlockSpec((_BLK, _COLS), lambda i: (i + blk0, 0)),
        ],
        out_specs=[
            pl.BlockSpec((1, 128), lambda i: (0, 0)),
            pl.BlockSpec((_BLK // 32, _COLS), lambda i: (i, 0)),
        ],
        out_shape=[
            jax.ShapeDtypeStruct((1, 128), jnp.float32),
            jax.ShapeDtypeStruct(((_ROWS - _R_SC) // 32, _COLS), jnp.int32),
        ],
    )(labels, mask)

    out_a = pl.pallas_call(
        _pass2a_body,
        grid=(blk0,),
        in_specs=[
            pl.BlockSpec((12, 128), lambda i: (0, 0)),
            pl.BlockSpec((1, 128), lambda i: (0, 0)),
            pl.BlockSpec((_BLK // 32, _COLS), lambda i: (i, 0)),
            pl.BlockSpec((_BLK, _COLS), lambda i: (i, 0)),
        ],
        out_specs=pl.BlockSpec((_BLK, _COLS), lambda i: (i, 0)),
        out_shape=jax.ShapeDtypeStruct((_ROWS, _COLS), jnp.float32),
    )(acc, tc_acc, packed_sc, mask)

    out = pl.pallas_call(
        _pass2b_body,
        grid=(tc_grid,),
        in_specs=[
            pl.BlockSpec((12, 128), lambda i: (0, 0)),
            pl.BlockSpec((1, 128), lambda i: (0, 0)),
            pl.BlockSpec((_BLK // 32, _COLS), lambda i: (i, 0)),
            pl.BlockSpec((_BLK, _COLS), lambda i: (i + blk0, 0)),
            pl.BlockSpec((8, 128), lambda i: (0, 0)),
        ],
        out_specs=pl.BlockSpec((_BLK, _COLS), lambda i: (i + blk0, 0)),
        out_shape=jax.ShapeDtypeStruct((_ROWS, _COLS), jnp.float32),
        input_output_aliases={4: 0},
    )(acc, tc_acc, packed, mask, out_a)
    return out


# speedup vs baseline: 1.0691x; 1.0144x over previous
import functools

import jax
import jax.numpy as jnp
from jax import lax
from jax.experimental import pallas as pl
from jax.experimental.pallas import tpu as pltpu
from jax.experimental.pallas import tpu_sc as plsc

_NUM_CLASSES = 2
_CLIPMIN = 0.05
_CLIPMAX = 0.95

_ROWS = 8192
_COLS = 4096
_TOTAL = _ROWS * _COLS

_NC = 2
_NS = 16
_NW = _NC * _NS
_R_SC = 2048
_ROWS_W = _R_SC // _NW
_GROUPS = _ROWS_W // 32
_CH_ROWS = 4
_NCH = _ROWS_W // _CH_ROWS
_GCH = 32 // _CH_ROWS
_VPR = _COLS // 16
_BLK = 512


def _sc_reduce_chunk(lab_bufs, m_bufs, pacc, k, accs):
    shift = 4 * k

    def inner(i, accs):
        new = []
        labs = []
        for r in range(_CH_ROWS):
            lab = lab_bufs[r][pl.ds(i * 16, 16)]
            m = m_bufs[r][pl.ds(i * 16, 16)]
            am, c1, cs = accs[r]
            selm = m > 0.0
            ones = jnp.where(selm, 1.0, 0.0)
            am = am + m
            cs = cs + ones
            c1 = c1 + ones * lab.astype(jnp.float32)
            labs.append(lab)
            new.append((am, c1, cs))
        nib = (labs[0] | (labs[1] << 1) | (labs[2] << 2) | (labs[3] << 3))
        pacc[pl.ds(i * 16, 16)] = pacc[pl.ds(i * 16, 16)] | (nib << shift)
        return tuple(new)

    return lax.fori_loop(0, _VPR, inner, accs)


def _sc_pass1_body(labels_hbm, mask_hbm, out_hbm, packed_hbm, lab_buf, m_buf,
                   outv, pacc, sl0, sl1, sm0, sm1):
    wid = lax.axis_index("s") * _NC + lax.axis_index("c")
    base = wid * _ROWS_W
    sem_l = (sl0, sl1)
    sem_m = (sm0, sm1)

    pltpu.async_copy(labels_hbm.at[pl.ds(base, _CH_ROWS)], lab_buf.at[0], sem_l[0])
    pltpu.async_copy(mask_hbm.at[pl.ds(base, _CH_ROWS)], m_buf.at[0], sem_m[0])

    zero_i = jnp.zeros((16,), jnp.int32)
    for g in range(_GROUPS):
        pacc_g = pacc.at[g]

        def zero_body(i, _, pacc_g=pacc_g):
            pacc_g[pl.ds(i * 16, 16)] = zero_i
            return 0

        lax.fori_loop(0, _VPR, zero_body, 0)

    zero = jnp.zeros((16,), jnp.float32)
    accs = tuple((zero, zero, zero) for _ in range(_CH_ROWS))

    for g in range(_GROUPS):
        pacc_g = pacc.at[g]

        def body2(p, accs, g=g, pacc_g=pacc_g):
            for b in range(2):
                kl = 2 * p + b
                k = g * _GCH + kl
                nb = 1 - b

                @pl.when(k + 1 < _NCH)
                def _start_next():
                    off = base + (k + 1) * _CH_ROWS
                    pltpu.async_copy(labels_hbm.at[pl.ds(off, _CH_ROWS)],
                                     lab_buf.at[nb], sem_l[nb])
                    pltpu.async_copy(mask_hbm.at[pl.ds(off, _CH_ROWS)],
                                     m_buf.at[nb], sem_m[nb])

                pltpu.make_async_copy(labels_hbm.at[pl.ds(0, _CH_ROWS)],
                                      lab_buf.at[b], sem_l[b]).wait()
                pltpu.make_async_copy(mask_hbm.at[pl.ds(0, _CH_ROWS)],
                                      m_buf.at[b], sem_m[b]).wait()
                accs = _sc_reduce_chunk(
                    [lab_buf.at[b, r] for r in range(_CH_ROWS)],
                    [m_buf.at[b, r] for r in range(_CH_ROWS)], pacc_g, kl, accs)
            return accs

        accs = lax.fori_loop(0, _GCH // 2, body2, accs)
        pltpu.sync_copy(pacc_g,
                        packed_hbm.at[pl.ds((wid * _GROUPS + g) * _COLS, _COLS)])

    am = accs[0][0] + accs[1][0] + accs[2][0] + accs[3][0]
    c1 = accs[0][1] + accs[1][1] + accs[2][1] + accs[3][1]
    cs = accs[0][2] + accs[1][2] + accs[2][2] + accs[3][2]
    outv[pl.ds(0, 16)] = am
    outv[pl.ds(16, 16)] = c1
    outv[pl.ds(32, 16)] = cs
    pltpu.sync_copy(outv, out_hbm.at[pl.ds(wid * 48, 48)])


_sc_pass1 = functools.partial(
    pl.kernel,
    mesh=plsc.VectorSubcoreMesh(core_axis_name="c", subcore_axis_name="s"),
    out_type=[
        jax.ShapeDtypeStruct((_NW * 48,), jnp.float32),
        jax.ShapeDtypeStruct((_R_SC // 32 * _COLS,), jnp.int32),
    ],
    scratch_types=[
        pltpu.VMEM((2, _CH_ROWS, _COLS), jnp.int32),
        pltpu.VMEM((2, _CH_ROWS, _COLS), jnp.float32),
        pltpu.VMEM((48,), jnp.float32),
        pltpu.VMEM((_GROUPS, _COLS), jnp.int32),
        pltpu.SemaphoreType.DMA,
        pltpu.SemaphoreType.DMA,
        pltpu.SemaphoreType.DMA,
        pltpu.SemaphoreType.DMA,
    ],
)(_sc_pass1_body)


def _tc_pass1_body(labels_ref, mask_ref, acc_ref, packed_ref):
    i = pl.program_id(0)

    @pl.when(i == 0)
    def _init():
        acc_ref[...] = jnp.zeros_like(acc_ref)

    m = mask_ref[...]
    lab = labels_ref[...]
    sel = (m > 0.0).astype(jnp.float32)
    s_mask = jnp.sum(m)
    c1 = jnp.sum(sel * lab.astype(jnp.float32))
    csel = jnp.sum(sel)
    lane = lax.broadcasted_iota(jnp.int32, (1, 128), 1)
    pv = (jnp.where(lane == 0, s_mask, 0.0)
          + jnp.where(lane == 1, c1, 0.0)
          + jnp.where(lane == 2, csel, 0.0))
    acc_ref[...] += pv
    lab3 = lab.reshape(_BLK // 32, 32, _COLS)
    k = lax.broadcasted_iota(jnp.int32, (_BLK // 32, 32, _COLS), 1)
    packed_ref[...] = jnp.sum(lab3 << k, axis=1)


def _weights(acc_ref, tc_acc_ref):
    acc = acc_ref[...]
    row = lax.broadcasted_iota(jnp.int32, (12, 128), 0)
    col = lax.broadcasted_iota(jnp.int32, (12, 128), 1)
    lane = (row * 128 + col) % 48
    tc_acc = tc_acc_ref[...]
    masked_in = jnp.sum(jnp.where(lane < 16, acc, 0.0)) + tc_acc[0, 0]
    c1 = jnp.sum(jnp.where((lane >= 16) & (lane < 32), acc, 0.0)) + tc_acc[0, 1]
    csel = jnp.sum(jnp.where(lane >= 32, acc, 0.0)) + tc_acc[0, 2]
    c0 = csel - c1

    inv_n = 1.0 / float(_NUM_CLASSES)

    def weight(c):
        frac = jnp.where(masked_in > 0.0, c / masked_in, 0.0)
        frac = jnp.clip(frac, _CLIPMIN, _CLIPMAX)
        w = inv_n / frac
        return jnp.where(c > 0.0, w, 0.0)

    return weight(c0), weight(c1)


def _unpacked_scale(packed_ref, mask_ref, out_ref, w0, w1):
    m = mask_ref[...]
    packed = packed_ref[...]
    p3 = jnp.broadcast_to(packed[:, None, :], (_BLK // 32, 32, _COLS))
    k = lax.broadcasted_iota(jnp.int32, (_BLK // 32, 32, _COLS), 1)
    lab = ((p3 >> k) & 1).reshape(_BLK, _COLS)
    out_ref[...] = m * jnp.where(lab == 1, w1, w0)


_BLKP = _R_SC // _BLK


def _pass2_body(acc_ref, tc_acc_ref, psc_ref, ptc_ref, mask_ref, out_ref):
    i = pl.program_id(0)
    w0, w1 = _weights(acc_ref, tc_acc_ref)

    @pl.when(i < _BLKP)
    def _sc_rows():
        _unpacked_scale(psc_ref, mask_ref, out_ref, w0, w1)

    @pl.when(i >= _BLKP)
    def _tc_rows():
        _unpacked_scale(ptc_ref, mask_ref, out_ref, w0, w1)


@jax.jit
def kernel(labels, mask):
    acc, packed_sc = _sc_pass1(labels, mask)
    acc = acc.reshape(12, 128)
    packed_sc = packed_sc.reshape(_R_SC // 32, _COLS)

    tc_grid = (_ROWS - _R_SC) // _BLK
    blk0 = _R_SC // _BLK
    tc_acc, packed = pl.pallas_call(
        _tc_pass1_body,
        grid=(tc_grid,),
        in_specs=[
            pl.BlockSpec((_BLK, _COLS), lambda i: (i + blk0, 0)),
            pl.BlockSpec((_BLK, _COLS), lambda i: (i + blk0, 0)),
        ],
        out_specs=[
            pl.BlockSpec((1, 128), lambda i: (0, 0)),
            pl.BlockSpec((_BLK // 32, _COLS), lambda i: (i, 0)),
        ],
        out_shape=[
            jax.ShapeDtypeStruct((1, 128), jnp.float32),
            jax.ShapeDtypeStruct(((_ROWS - _R_SC) // 32, _COLS), jnp.int32),
        ],
    )(labels, mask)

    out = pl.pallas_call(
        _pass2_body,
        grid=(_ROWS // _BLK,),
        in_specs=[
            pl.BlockSpec((12, 128), lambda i: (0, 0)),
            pl.BlockSpec((1, 128), lambda i: (0, 0)),
            pl.BlockSpec((_BLK // 32, _COLS),
                         lambda i: (jnp.minimum(i, _BLKP - 1), 0)),
            pl.BlockSpec((_BLK // 32, _COLS),
                         lambda i: (jnp.maximum(i - _BLKP, 0), 0)),
            pl.BlockSpec((_BLK, _COLS), lambda i: (i, 0)),
        ],
        out_specs=pl.BlockSpec((_BLK, _COLS), lambda i: (i, 0)),
        out_shape=jax.ShapeDtypeStruct((_ROWS, _COLS), jnp.float32),
    )(acc, tc_acc, packed_sc, packed, mask)
    return out


# speedup vs baseline: 1.0691x; 1.0000x over previous
import functools

import jax
import jax.numpy as jnp
from jax import lax
from jax.experimental import pallas as pl
from jax.experimental.pallas import tpu as pltpu
from jax.experimental.pallas import tpu_sc as plsc

_NUM_CLASSES = 2
_CLIPMIN = 0.05
_CLIPMAX = 0.95

_ROWS = 8192
_COLS = 4096
_TOTAL = _ROWS * _COLS

_NC = 2
_NS = 16
_NW = _NC * _NS
_R_SC = 3072
_ROWS_W = _R_SC // _NW
_GROUPS = _ROWS_W // 32
_CH_ROWS = 4
_NCH = _ROWS_W // _CH_ROWS
_GCH = 32 // _CH_ROWS
_VPR = _COLS // 16
_BLK = 512


def _sc_reduce_chunk(lab_bufs, m_bufs, pacc, k, accs):
    shift = 4 * k

    def inner(i, accs):
        new = []
        labs = []
        for r in range(_CH_ROWS):
            lab = lab_bufs[r][pl.ds(i * 16, 16)]
            m = m_bufs[r][pl.ds(i * 16, 16)]
            am, c1, cs = accs[r]
            selm = m > 0.0
            ones = jnp.where(selm, 1.0, 0.0)
            am = am + m
            cs = cs + ones
            c1 = c1 + ones * lab.astype(jnp.float32)
            labs.append(lab)
            new.append((am, c1, cs))
        nib = (labs[0] | (labs[1] << 1) | (labs[2] << 2) | (labs[3] << 3))
        pacc[pl.ds(i * 16, 16)] = pacc[pl.ds(i * 16, 16)] | (nib << shift)
        return tuple(new)

    return lax.fori_loop(0, _VPR, inner, accs)


def _sc_pass1_body(labels_hbm, mask_hbm, out_hbm, packed_hbm, lab_buf, m_buf,
                   outv, pacc, sl0, sl1, sm0, sm1):
    wid = lax.axis_index("s") * _NC + lax.axis_index("c")
    base = wid * _ROWS_W
    sem_l = (sl0, sl1)
    sem_m = (sm0, sm1)

    pltpu.async_copy(labels_hbm.at[pl.ds(base, _CH_ROWS)], lab_buf.at[0], sem_l[0])
    pltpu.async_copy(mask_hbm.at[pl.ds(base, _CH_ROWS)], m_buf.at[0], sem_m[0])

    zero_i = jnp.zeros((16,), jnp.int32)
    for g in range(_GROUPS):
        pacc_g = pacc.at[g]

        def zero_body(i, _, pacc_g=pacc_g):
            pacc_g[pl.ds(i * 16, 16)] = zero_i
            return 0

        lax.fori_loop(0, _VPR, zero_body, 0)

    zero = jnp.zeros((16,), jnp.float32)
    accs = tuple((zero, zero, zero) for _ in range(_CH_ROWS))

    for g in range(_GROUPS):
        pacc_g = pacc.at[g]

        def body2(p, accs, g=g, pacc_g=pacc_g):
            for b in range(2):
                kl = 2 * p + b
                k = g * _GCH + kl
                nb = 1 - b

                @pl.when(k + 1 < _NCH)
                def _start_next():
                    off = base + (k + 1) * _CH_ROWS
                    pltpu.async_copy(labels_hbm.at[pl.ds(off, _CH_ROWS)],
                                     lab_buf.at[nb], sem_l[nb])
                    pltpu.async_copy(mask_hbm.at[pl.ds(off, _CH_ROWS)],
                                     m_buf.at[nb], sem_m[nb])

                pltpu.make_async_copy(labels_hbm.at[pl.ds(0, _CH_ROWS)],
                                      lab_buf.at[b], sem_l[b]).wait()
                pltpu.make_async_copy(mask_hbm.at[pl.ds(0, _CH_ROWS)],
                                      m_buf.at[b], sem_m[b]).wait()
                accs = _sc_reduce_chunk(
                    [lab_buf.at[b, r] for r in range(_CH_ROWS)],
                    [m_buf.at[b, r] for r in range(_CH_ROWS)], pacc_g, kl, accs)
            return accs

        accs = lax.fori_loop(0, _GCH // 2, body2, accs)
        pltpu.sync_copy(pacc_g,
                        packed_hbm.at[pl.ds((wid * _GROUPS + g) * _COLS, _COLS)])

    am = accs[0][0] + accs[1][0] + accs[2][0] + accs[3][0]
    c1 = accs[0][1] + accs[1][1] + accs[2][1] + accs[3][1]
    cs = accs[0][2] + accs[1][2] + accs[2][2] + accs[3][2]
    outv[pl.ds(0, 16)] = am
    outv[pl.ds(16, 16)] = c1
    outv[pl.ds(32, 16)] = cs
    pltpu.sync_copy(outv, out_hbm.at[pl.ds(wid * 48, 48)])


_sc_pass1 = functools.partial(
    pl.kernel,
    mesh=plsc.VectorSubcoreMesh(core_axis_name="c", subcore_axis_name="s"),
    out_type=[
        jax.ShapeDtypeStruct((_NW * 48,), jnp.float32),
        jax.ShapeDtypeStruct((_R_SC // 32 * _COLS,), jnp.int32),
    ],
    scratch_types=[
        pltpu.VMEM((2, _CH_ROWS, _COLS), jnp.int32),
        pltpu.VMEM((2, _CH_ROWS, _COLS), jnp.float32),
        pltpu.VMEM((48,), jnp.float32),
        pltpu.VMEM(((_GROUPS + 1) // 2 * 2, _COLS), jnp.int32),
        pltpu.SemaphoreType.DMA,
        pltpu.SemaphoreType.DMA,
        pltpu.SemaphoreType.DMA,
        pltpu.SemaphoreType.DMA,
    ],
)(_sc_pass1_body)


def _tc_pass1_body(labels_ref, mask_ref, acc_ref, packed_ref):
    i = pl.program_id(0)

    @pl.when(i == 0)
    def _init():
        acc_ref[...] = jnp.zeros_like(acc_ref)

    m = mask_ref[...]
    lab = labels_ref[...]
    sel = (m > 0.0).astype(jnp.float32)
    s_mask = jnp.sum(m)
    c1 = jnp.sum(sel * lab.astype(jnp.float32))
    csel = jnp.sum(sel)
    lane = lax.broadcasted_iota(jnp.int32, (1, 128), 1)
    pv = (jnp.where(lane == 0, s_mask, 0.0)
          + jnp.where(lane == 1, c1, 0.0)
          + jnp.where(lane == 2, csel, 0.0))
    acc_ref[...] += pv
    lab3 = lab.reshape(_BLK // 32, 32, _COLS)
    k = lax.broadcasted_iota(jnp.int32, (_BLK // 32, 32, _COLS), 1)
    packed_ref[...] = jnp.sum(lab3 << k, axis=1)


def _weights(acc_ref, tc_acc_ref):
    acc = acc_ref[...]
    row = lax.broadcasted_iota(jnp.int32, (12, 128), 0)
    col = lax.broadcasted_iota(jnp.int32, (12, 128), 1)
    lane = (row * 128 + col) % 48
    tc_acc = tc_acc_ref[...]
    masked_in = jnp.sum(jnp.where(lane < 16, acc, 0.0)) + tc_acc[0, 0]
    c1 = jnp.sum(jnp.where((lane >= 16) & (lane < 32), acc, 0.0)) + tc_acc[0, 1]
    csel = jnp.sum(jnp.where(lane >= 32, acc, 0.0)) + tc_acc[0, 2]
    c0 = csel - c1

    inv_n = 1.0 / float(_NUM_CLASSES)

    def weight(c):
        frac = jnp.where(masked_in > 0.0, c / masked_in, 0.0)
        frac = jnp.clip(frac, _CLIPMIN, _CLIPMAX)
        w = inv_n / frac
        return jnp.where(c > 0.0, w, 0.0)

    return weight(c0), weight(c1)


def _unpacked_scale(packed_ref, mask_ref, out_ref, w0, w1):
    m = mask_ref[...]
    packed = packed_ref[...]
    p3 = jnp.broadcast_to(packed[:, None, :], (_BLK // 32, 32, _COLS))
    k = lax.broadcasted_iota(jnp.int32, (_BLK // 32, 32, _COLS), 1)
    lab = ((p3 >> k) & 1).reshape(_BLK, _COLS)
    out_ref[...] = m * jnp.where(lab == 1, w1, w0)


_BLKP = _R_SC // _BLK


def _pass2_body(acc_ref, tc_acc_ref, psc_ref, ptc_ref, mask_ref, out_ref):
    i = pl.program_id(0)
    w0, w1 = _weights(acc_ref, tc_acc_ref)

    @pl.when(i < _BLKP)
    def _sc_rows():
        _unpacked_scale(psc_ref, mask_ref, out_ref, w0, w1)

    @pl.when(i >= _BLKP)
    def _tc_rows():
        _unpacked_scale(ptc_ref, mask_ref, out_ref, w0, w1)


@jax.jit
def kernel(labels, mask):
    acc, packed_sc = _sc_pass1(labels, mask)
    acc = acc.reshape(12, 128)
    packed_sc = packed_sc.reshape(_R_SC // 32, _COLS)

    tc_grid = (_ROWS - _R_SC) // _BLK
    blk0 = _R_SC // _BLK
    tc_acc, packed = pl.pallas_call(
        _tc_pass1_body,
        grid=(tc_grid,),
        in_specs=[
            pl.BlockSpec((_BLK, _COLS), lambda i: (i + blk0, 0)),
            pl.BlockSpec((_BLK, _COLS), lambda i: (i + blk0, 0)),
        ],
        out_specs=[
            pl.BlockSpec((1, 128), lambda i: (0, 0)),
            pl.BlockSpec((_BLK // 32, _COLS), lambda i: (i, 0)),
        ],
        out_shape=[
            jax.ShapeDtypeStruct((1, 128), jnp.float32),
            jax.ShapeDtypeStruct(((_ROWS - _R_SC) // 32, _COLS), jnp.int32),
        ],
    )(labels, mask)

    out = pl.pallas_call(
        _pass2_body,
        grid=(_ROWS // _BLK,),
        in_specs=[
            pl.BlockSpec((12, 128), lambda i: (0, 0)),
            pl.BlockSpec((1, 128), lambda i: (0, 0)),
            pl.BlockSpec((_BLK // 32, _COLS),
                         lambda i: (jnp.minimum(i, _BLKP - 1), 0)),
            pl.BlockSpec((_BLK // 32, _COLS),
                         lambda i: (jnp.maximum(i - _BLKP, 0), 0)),
            pl.BlockSpec((_BLK, _COLS), lambda i: (i, 0)),
        ],
        out_specs=pl.BlockSpec((_BLK, _COLS), lambda i: (i, 0)),
        out_shape=jax.ShapeDtypeStruct((_ROWS, _COLS), jnp.float32),
    )(acc, tc_acc, packed_sc, packed, mask)
    return out
